# baseline (speedup 1.0000x reference)
import functools

import jax
import jax.numpy as jnp
from jax import lax
from jax.experimental import pallas as pl
from jax.experimental.pallas import tpu as pltpu
from jax.experimental.pallas import tpu_sc as plsc

HIST = 200
D = 16
NSUB = 16
PER_W = 16
NPAD = NSUB * PER_W


def _ehr_sc(idx_pad, emb3, w16, b16):
    mesh = plsc.VectorSubcoreMesh(core_axis_name="c", subcore_axis_name="s")

    @functools.partial(
        pl.kernel,
        mesh=mesh,
        compiler_params=pltpu.CompilerParams(needs_layout_passes=False),
        out_type=[
            jax.ShapeDtypeStruct((D,), jnp.float32),
            jax.ShapeDtypeStruct((D,), jnp.float32),
            jax.ShapeDtypeStruct((NSUB, 8 * D), jnp.float32),
        ],
        scratch_types=[
            pltpu.VMEM((PER_W,), jnp.int32),
            pltpu.VMEM((PER_W, 8 * D), jnp.float32),
            pltpu.VMEM((D,), jnp.float32),
            pltpu.VMEM((D,), jnp.float32),
            pltpu.VMEM((D,), jnp.float32),
            pltpu.VMEM((NSUB, 8 * D), jnp.float32),
            pltpu.SemaphoreType.DMA,
        ],
    )
    def k(idx_hbm, emb_hbm, w_hbm, b_hbm, emb_out, sig_out, part_out,
          idx_v, rows_v, vec_v, w_v, b_v, all_v, sem):
        cid = lax.axis_index("c")
        sid = lax.axis_index("s")

        @pl.when(cid == 0)
        def _():
            base = sid * PER_W
            pltpu.sync_copy(idx_hbm.at[pl.ds(base, PER_W)], idx_v)
            ids = idx_v[...]
            hi = lax.shift_right_logical(ids, 3)
            lo = jnp.bitwise_and(ids, 7)
            pltpu.async_copy(emb_hbm.at[hi], rows_v, sem).wait()
            lane = lax.iota(jnp.int32, PER_W)
            validf = jnp.clip(HIST - base - lane, 0, 1).astype(jnp.float32)
            col = lo * D
            vec_v[...] = jnp.zeros((D,), jnp.float32)
            for dcomp in range(D):
                vals = plsc.load_gather(rows_v, [lane, col + dcomp])
                plsc.addupdate_scatter(
                    vec_v, [jnp.full((PER_W,), dcomp, jnp.int32)],
                    vals * validf)
            pltpu.sync_copy(vec_v, part_out.at[sid, pl.ds(0, D)])

        plsc.subcore_barrier()

        @pl.when(jnp.logical_and(cid == 0, sid == 0))
        def _():
            rid = lax.iota(jnp.int32, NSUB)
            pltpu.async_copy(part_out.at[rid], all_v, sem).wait()
            lane = lax.iota(jnp.int32, D)
            parts = [
                plsc.load_gather(all_v, [jnp.full((D,), i, jnp.int32), lane])
                for i in range(NSUB)
            ]
            while len(parts) > 1:
                parts = [parts[i] + parts[i + 1]
                         for i in range(0, len(parts), 2)]
            acc = parts[0]
            vec_v[...] = acc
            pltpu.sync_copy(vec_v, emb_out)
            pltpu.sync_copy(w_hbm, w_v)
            pltpu.sync_copy(b_hbm, b_v)
            prod = acc * w_v[...]
            x = jnp.full((D,), jnp.sum(prod), jnp.float32) + b_v[...]
            vec_v[...] = 1.0 / (1.0 + jnp.exp(-x))
            pltpu.sync_copy(vec_v, sig_out)

    return k(idx_pad, emb3, w16, b16)


def kernel(label, ehr_seq, emb, W, b):
    idx = ehr_seq.astype(jnp.int32)
    idx_pad = jnp.pad(idx, (0, NPAD - HIST))
    emb3 = emb.reshape(emb.shape[0] // 8, 8 * D)
    w16 = W.reshape(D).astype(jnp.float32)
    b16 = jnp.broadcast_to(b.astype(jnp.float32), (D,))
    pooled, sig, _ = _ehr_sc(idx_pad, emb3, w16, b16)
    embedded = pooled.reshape(1, D)
    output = sig[:1].reshape(1, 1)
    return (output, label, embedded)
